# baseline (speedup 1.0000x reference)
import functools

import jax
import jax.numpy as jnp
from jax import lax
from jax.experimental import pallas as pl
from jax.experimental.pallas import tpu as pltpu
from jax.experimental.pallas import tpu_sc as plsc

VOCAB = 32000
CHAR_VOCAB = 6000
MAX_CHARS = 16
D_CHAR = 128
D_MODEL = 768

N_TOK = 4 * 4096
N_WORKERS = 32
TOK_PER_W = N_TOK // N_WORKERS
NBUF = 8
SG = NBUF
N_SG = TOK_PER_W // SG
AUG = 160
LN_LANE = 5
SUB = D_CHAR // 16


def _sc_pool(tok_hbm, aux_hbm, table_hbm, out_hbm,
             tok_v, aux_v, gidx, rows, out_v, sems, sem2):
    cid = lax.axis_index("c")
    sid = lax.axis_index("s")
    wid = sid * 2 + cid
    base = wid * TOK_PER_W
    iota = lax.iota(jnp.int32, 16)

    pltpu.sync_copy(tok_hbm.at[pl.ds(base, TOK_PER_W)], tok_v)
    cps = [pltpu.async_copy(
        aux_hbm.at[tok_v.at[pl.ds(c * 128, 128)]],
        aux_v.at[pl.ds(c * 128, 128)], sem2)
        for c in range(TOK_PER_W // 128)]
    for cp in cps:
        cp.wait()

    def super_group(i, carry):
        cps = []
        for b in range(NBUF):
            tt = i * SG + b
            raw = aux_v[tt, pl.ds(0, 16)]
            cids = raw & 0xFFFF
            ln = lax.shift_right_logical(raw, 16)[0]
            mcids = jnp.where(iota < ln, cids, CHAR_VOCAB)
            for k in range(SUB):
                gidx[b][pl.ds(k * 16, 16)] = mcids * SUB + k
            cps.append(pltpu.async_copy(
                table_hbm.at[gidx[b]], rows[b], sems[b]))
        for b in range(NBUF):
            cps[b].wait()
            tt = i * SG + b
            ex = lax.shift_right_logical(aux_v[tt, pl.ds(0, 16)], 16)
            for k in range(SUB):
                acc = rows[b][k * 16, pl.ds(0, 16)]
                for j in range(1, MAX_CHARS):
                    acc = acc + rows[b][k * 16 + j, pl.ds(0, 16)]
                out_v[b, pl.ds(16 * k, 16)] = acc
            lnf = ex[0].astype(jnp.float32)
            tail = jnp.where(
                (iota == ex[1]) | (iota == ex[2] + 3) | (iota == LN_LANE),
                lnf, jnp.float32(0.0))
            out_v[b, pl.ds(128, 16)] = tail
            out_v[b, pl.ds(144, 16)] = jnp.zeros((16,), jnp.float32)
        pltpu.sync_copy(out_v, out_hbm.at[pl.ds(base + i * SG, SG)])
        return carry

    lax.fori_loop(0, N_SG, super_group, 0)


_sc_pool_call = functools.partial(
    pl.kernel,
    out_type=jax.ShapeDtypeStruct((N_TOK, AUG), jnp.float32),
    mesh=plsc.VectorSubcoreMesh(core_axis_name="c", subcore_axis_name="s"),
    compiler_params=pltpu.CompilerParams(use_tc_tiling_on_sc=False),
    scratch_types=[
        pltpu.VMEM((TOK_PER_W,), jnp.int32),
        pltpu.VMEM((TOK_PER_W, 16), jnp.int32),
        [pltpu.VMEM((MAX_CHARS * SUB,), jnp.int32) for _ in range(NBUF)],
        [pltpu.VMEM((MAX_CHARS * SUB, 16), jnp.float32) for _ in range(NBUF)],
        pltpu.VMEM((SG, AUG), jnp.float32),
        [pltpu.SemaphoreType.DMA for _ in range(NBUF)],
        pltpu.SemaphoreType.DMA,
    ],
)(_sc_pool)


def _mm_body(x_ref, w_ref, o_ref):
    x = x_ref[...]
    y = jnp.dot(x, w_ref[...], preferred_element_type=jnp.float32)
    o_ref[...] = y / x[:, 128 + LN_LANE:128 + LN_LANE + 1]


def _project(pooled_aug, w_aug):
    bm = 256
    return pl.pallas_call(
        _mm_body,
        grid=(N_TOK // bm,),
        in_specs=[
            pl.BlockSpec((bm, AUG), lambda i: (i, 0)),
            pl.BlockSpec((AUG, D_MODEL), lambda i: (0, 0)),
        ],
        out_specs=pl.BlockSpec((bm, D_MODEL), lambda i: (i, 0)),
        out_shape=jax.ShapeDtypeStruct((N_TOK, D_MODEL), jnp.float32),
    )(pooled_aug, w_aug)


def kernel(token_ids, char_ids, char_lengths, char_table, proj_W,
           special_flags, special_emb, is_continuation, cont_emb):
    tok = token_ids.reshape(-1).astype(jnp.int32)
    nrows = char_ids.shape[0]
    hi = jnp.zeros((nrows, 16), jnp.int32)
    hi = hi.at[:, 0].set(char_lengths.astype(jnp.int32))
    hi = hi.at[:, 1].set(special_flags.astype(jnp.int32))
    hi = hi.at[:, 2].set(is_continuation.astype(jnp.int32))
    aux = char_ids.astype(jnp.int32) | (hi << 16)
    table_sub = char_table.at[CHAR_VOCAB].set(0.0).reshape(-1, 16)
    w_aug = jnp.concatenate(
        [proj_W, special_emb, cont_emb,
         jnp.zeros((AUG - D_CHAR - 5, D_MODEL), jnp.float32)], axis=0)

    pooled_aug = _sc_pool_call(tok, aux, table_sub)
    out = _project(pooled_aug, w_aug)
    return out.reshape(token_ids.shape[0], token_ids.shape[1], D_MODEL)


# speedup vs baseline: 1.5402x; 1.5402x over previous
import functools

import numpy as np
import jax
import jax.numpy as jnp
from jax import lax
from jax.experimental import pallas as pl
from jax.experimental.pallas import tpu as pltpu
from jax.experimental.pallas import tpu_sc as plsc

VOCAB = 32000
CHAR_VOCAB = 6000
MAX_CHARS = 16
D_CHAR = 128
D_MODEL = 768

N_TOK = 4 * 4096
N_WORKERS = 32
TOK_PER_W = N_TOK // N_WORKERS
NBUF = 8
SG = NBUF
N_SG = TOK_PER_W // SG
AUG = 160
LN_LANE = 5
SUB = D_CHAR // 32

_PERM = np.empty((D_CHAR,), np.int64)
for _k in range(SUB):
    for _i in range(16):
        _PERM[32 * _k + _i] = 32 * _k + 2 * _i
        _PERM[32 * _k + 16 + _i] = 32 * _k + 2 * _i + 1


def _sc_pool(tok_hbm, aux_hbm, table_hbm, out_hbm,
             tok_v, aux_v, gidx, rows, out_v, sems, sem2):
    cid = lax.axis_index("c")
    sid = lax.axis_index("s")
    wid = sid * 2 + cid
    base = wid * TOK_PER_W
    iota = lax.iota(jnp.int32, 16)

    pltpu.sync_copy(tok_hbm.at[pl.ds(base, TOK_PER_W)], tok_v)
    cps = [pltpu.async_copy(
        aux_hbm.at[tok_v.at[pl.ds(c * 128, 128)]],
        aux_v.at[pl.ds(c * 128, 128)], sem2)
        for c in range(TOK_PER_W // 128)]
    for cp in cps:
        cp.wait()

    def super_group(i, carry):
        cps = []
        for b in range(NBUF):
            tt = i * SG + b
            raw = aux_v[tt, pl.ds(0, 16)]
            cids = raw & 0xFFFF
            ln = lax.shift_right_logical(raw, 16)[0]
            mcids = jnp.where(iota < ln, cids, CHAR_VOCAB)
            for k in range(SUB):
                gidx[b][pl.ds(k * 16, 16)] = mcids * SUB + k
            cps.append(pltpu.async_copy(
                table_hbm.at[gidx[b]], rows[b], sems[b]))
        for b in range(NBUF):
            cps[b].wait()
            tt = i * SG + b
            ex = lax.shift_right_logical(aux_v[tt, pl.ds(0, 16)], 16)
            for k in range(SUB):
                acc_e = jnp.zeros((16,), jnp.float32)
                acc_o = jnp.zeros((16,), jnp.float32)
                for j in range(MAX_CHARS):
                    v = rows[b][k * 16 + j, pl.ds(0, 16)]
                    acc_e = acc_e + lax.bitcast_convert_type(
                        v << 16, jnp.float32)
                    acc_o = acc_o + lax.bitcast_convert_type(
                        v & -65536, jnp.float32)
                out_v[b, pl.ds(32 * k, 16)] = acc_e
                out_v[b, pl.ds(32 * k + 16, 16)] = acc_o
            lnf = ex[0].astype(jnp.float32)
            tail = jnp.where(
                (iota == ex[1]) | (iota == ex[2] + 3) | (iota == LN_LANE),
                lnf, jnp.float32(0.0))
            out_v[b, pl.ds(128, 16)] = tail
            out_v[b, pl.ds(144, 16)] = jnp.zeros((16,), jnp.float32)
        pltpu.sync_copy(out_v, out_hbm.at[pl.ds(base + i * SG, SG)])
        return carry

    lax.fori_loop(0, N_SG, super_group, 0)


_sc_pool_call = functools.partial(
    pl.kernel,
    out_type=jax.ShapeDtypeStruct((N_TOK, AUG), jnp.float32),
    mesh=plsc.VectorSubcoreMesh(core_axis_name="c", subcore_axis_name="s"),
    compiler_params=pltpu.CompilerParams(use_tc_tiling_on_sc=False),
    scratch_types=[
        pltpu.VMEM((TOK_PER_W,), jnp.int32),
        pltpu.VMEM((TOK_PER_W, 16), jnp.int32),
        [pltpu.VMEM((MAX_CHARS * SUB,), jnp.int32) for _ in range(NBUF)],
        [pltpu.VMEM((MAX_CHARS * SUB, 16), jnp.int32) for _ in range(NBUF)],
        pltpu.VMEM((SG, AUG), jnp.float32),
        [pltpu.SemaphoreType.DMA for _ in range(NBUF)],
        pltpu.SemaphoreType.DMA,
    ],
)(_sc_pool)


def _mm_body(x_ref, w_ref, o_ref):
    x = x_ref[...]
    y = jnp.dot(x, w_ref[...], preferred_element_type=jnp.float32)
    o_ref[...] = y / x[:, 128 + LN_LANE:128 + LN_LANE + 1]


def _project(pooled_aug, w_aug):
    bm = 256
    return pl.pallas_call(
        _mm_body,
        grid=(N_TOK // bm,),
        in_specs=[
            pl.BlockSpec((bm, AUG), lambda i: (i, 0)),
            pl.BlockSpec((AUG, D_MODEL), lambda i: (0, 0)),
        ],
        out_specs=pl.BlockSpec((bm, D_MODEL), lambda i: (i, 0)),
        out_shape=jax.ShapeDtypeStruct((N_TOK, D_MODEL), jnp.float32),
    )(pooled_aug, w_aug)


def kernel(token_ids, char_ids, char_lengths, char_table, proj_W,
           special_flags, special_emb, is_continuation, cont_emb):
    tok = token_ids.reshape(-1).astype(jnp.int32)
    nrows = char_ids.shape[0]
    hi = jnp.zeros((nrows, 16), jnp.int32)
    hi = hi.at[:, 0].set(char_lengths.astype(jnp.int32))
    hi = hi.at[:, 1].set(special_flags.astype(jnp.int32))
    hi = hi.at[:, 2].set(is_continuation.astype(jnp.int32))
    aux = char_ids.astype(jnp.int32) | (hi << 16)
    table_z = char_table.at[CHAR_VOCAB].set(0.0)
    table_sub = jax.lax.bitcast_convert_type(
        table_z.astype(jnp.bfloat16).reshape(-1, 16, 2), jnp.int32)
    w_aug = jnp.concatenate(
        [proj_W[jnp.asarray(_PERM)], special_emb, cont_emb,
         jnp.zeros((AUG - D_CHAR - 5, D_MODEL), jnp.float32)], axis=0)

    pooled_aug = _sc_pool_call(tok, aux, table_sub)
    out = _project(pooled_aug, w_aug)
    return out.reshape(token_ids.shape[0], token_ids.shape[1], D_MODEL)


# speedup vs baseline: 1.8871x; 1.2252x over previous
import functools

import numpy as np
import jax
import jax.numpy as jnp
from jax import lax
from jax.experimental import pallas as pl
from jax.experimental.pallas import tpu as pltpu
from jax.experimental.pallas import tpu_sc as plsc

VOCAB = 32000
CHAR_VOCAB = 6000
MAX_CHARS = 16
D_CHAR = 128
D_MODEL = 768

N_TOK = 4 * 4096
N_WORKERS = 32
TOK_PER_W = N_TOK // N_WORKERS
NBUF = 8
SG = NBUF
N_SG = TOK_PER_W // SG
AUG = 160
LN_LANE = 5
SUB = D_CHAR // 32

_PERM = np.empty((D_CHAR,), np.int64)
for _k in range(SUB):
    for _i in range(16):
        _PERM[32 * _k + _i] = 32 * _k + 2 * _i
        _PERM[32 * _k + 16 + _i] = 32 * _k + 2 * _i + 1


def _sc_pool(tok_hbm, aux_hbm, table_hbm, out_hbm,
             tok_v, aux_v, gidx, rows, out_v, sems, sem2):
    cid = lax.axis_index("c")
    sid = lax.axis_index("s")
    wid = sid * 2 + cid
    base = wid * TOK_PER_W
    iota = lax.iota(jnp.int32, 16)

    pltpu.sync_copy(tok_hbm.at[pl.ds(base, TOK_PER_W)], tok_v)
    cps = [pltpu.async_copy(
        aux_hbm.at[tok_v.at[pl.ds(c * 128, 128)]],
        aux_v.at[pl.ds(c * 128, 128)], sem2)
        for c in range(TOK_PER_W // 128)]
    for cp in cps:
        cp.wait()

    def super_group(i, carry):
        cps = []
        for b in range(NBUF):
            tt = i * SG + b
            raw = aux_v[tt, pl.ds(0, 16)]
            cids = raw & 0xFFFF
            ln = lax.shift_right_logical(raw, 16)[0]
            gidx[b][pl.ds(0, 16)] = jnp.where(iota < ln, cids, CHAR_VOCAB)
            cps.append(pltpu.async_copy(
                table_hbm.at[gidx[b]], rows[b], sems[b]))
        for b in range(NBUF):
            cps[b].wait()
            tt = i * SG + b
            ex = lax.shift_right_logical(aux_v[tt, pl.ds(0, 16)], 16)
            for k in range(SUB):
                acc_e = jnp.zeros((16,), jnp.float32)
                acc_o = jnp.zeros((16,), jnp.float32)
                for j in range(MAX_CHARS):
                    v = rows[b][j, pl.ds(k * 16, 16)]
                    acc_e = acc_e + lax.bitcast_convert_type(
                        v << 16, jnp.float32)
                    acc_o = acc_o + lax.bitcast_convert_type(
                        v & -65536, jnp.float32)
                out_v[b, pl.ds(32 * k, 16)] = acc_e
                out_v[b, pl.ds(32 * k + 16, 16)] = acc_o
            lnf = ex[0].astype(jnp.float32)
            tail = jnp.where(
                (iota == ex[1]) | (iota == ex[2] + 3) | (iota == LN_LANE),
                lnf, jnp.float32(0.0))
            out_v[b, pl.ds(128, 16)] = tail
            out_v[b, pl.ds(144, 16)] = jnp.zeros((16,), jnp.float32)
        pltpu.sync_copy(out_v, out_hbm.at[pl.ds(base + i * SG, SG)])
        return carry

    lax.fori_loop(0, N_SG, super_group, 0)


_sc_pool_call = functools.partial(
    pl.kernel,
    out_type=jax.ShapeDtypeStruct((N_TOK, AUG), jnp.float32),
    mesh=plsc.VectorSubcoreMesh(core_axis_name="c", subcore_axis_name="s"),
    compiler_params=pltpu.CompilerParams(use_tc_tiling_on_sc=False),
    scratch_types=[
        pltpu.VMEM((TOK_PER_W,), jnp.int32),
        pltpu.VMEM((TOK_PER_W, 16), jnp.int32),
        [pltpu.VMEM((16,), jnp.int32) for _ in range(NBUF)],
        [pltpu.VMEM((MAX_CHARS, SUB * 16), jnp.int32) for _ in range(NBUF)],
        pltpu.VMEM((SG, AUG), jnp.float32),
        [pltpu.SemaphoreType.DMA for _ in range(NBUF)],
        pltpu.SemaphoreType.DMA,
    ],
)(_sc_pool)


def _mm_body(x_ref, w_ref, o_ref):
    x = x_ref[...]
    y = jnp.dot(x, w_ref[...], preferred_element_type=jnp.float32)
    o_ref[...] = y / x[:, 128 + LN_LANE:128 + LN_LANE + 1]


def _project(pooled_aug, w_aug):
    bm = 256
    return pl.pallas_call(
        _mm_body,
        grid=(N_TOK // bm,),
        in_specs=[
            pl.BlockSpec((bm, AUG), lambda i: (i, 0)),
            pl.BlockSpec((AUG, D_MODEL), lambda i: (0, 0)),
        ],
        out_specs=pl.BlockSpec((bm, D_MODEL), lambda i: (i, 0)),
        out_shape=jax.ShapeDtypeStruct((N_TOK, D_MODEL), jnp.float32),
    )(pooled_aug, w_aug)


def kernel(token_ids, char_ids, char_lengths, char_table, proj_W,
           special_flags, special_emb, is_continuation, cont_emb):
    tok = token_ids.reshape(-1).astype(jnp.int32)
    nrows = char_ids.shape[0]
    hi = jnp.zeros((nrows, 16), jnp.int32)
    hi = hi.at[:, 0].set(char_lengths.astype(jnp.int32))
    hi = hi.at[:, 1].set(special_flags.astype(jnp.int32))
    hi = hi.at[:, 2].set(is_continuation.astype(jnp.int32))
    aux = char_ids.astype(jnp.int32) | (hi << 16)
    table_z = char_table.at[CHAR_VOCAB].set(0.0)
    table_sub = jax.lax.bitcast_convert_type(
        table_z.astype(jnp.bfloat16).reshape(-1, SUB * 16, 2), jnp.int32)
    w_aug = jnp.concatenate(
        [proj_W[jnp.asarray(_PERM)], special_emb, cont_emb,
         jnp.zeros((AUG - D_CHAR - 5, D_MODEL), jnp.float32)], axis=0)

    pooled_aug = _sc_pool_call(tok, aux, table_sub)
    out = _project(pooled_aug, w_aug)
    return out.reshape(token_ids.shape[0], token_ids.shape[1], D_MODEL)
